# baseline (speedup 1.0000x reference)
import functools

import jax
import jax.numpy as jnp
from jax import lax
from jax.experimental import pallas as pl
from jax.experimental.pallas import tpu as pltpu
from jax.experimental.pallas import tpu_sc as plsc

EMBED_DIM = 1024
PAD = 1
BATCH = 4
SEQ = 8192
NROWS = BATCH * SEQ
NWORKERS = 32
SPW = SEQ // NWORKERS
CHUNK = 32
NCHUNK = SPW // CHUNK
NBUF = 3
LANES = 16
KFIX = 8

_mesh = plsc.VectorSubcoreMesh(core_axis_name="c", subcore_axis_name="s")


@functools.partial(
    pl.kernel,
    mesh=_mesh,
    out_type=jax.ShapeDtypeStruct((NROWS, EMBED_DIM), jnp.float32),
    scratch_types=[
        pltpu.VMEM((SPW,), jnp.int32),
        pltpu.VMEM((NBUF * CHUNK, EMBED_DIM), jnp.float32),
        pltpu.SemaphoreType.DMA((NBUF,)),
        pltpu.SemaphoreType.DMA((NBUF,)),
    ],
)
def _emb_stream(w_hbm, out_hbm, idx_v, rows, gsems, wsems):
    wid = lax.axis_index("s") * 2 + lax.axis_index("c")
    seq0 = wid * SPW

    def compute_idx(j, carry):
        off = j * LANES
        idx_v[pl.ds(off, LANES)] = (
            seq0 + off + (PAD + 1) + lax.iota(jnp.int32, LANES))
        return carry

    lax.fori_loop(0, SPW // LANES, compute_idx, 0)

    def slot(c):
        return c - (c // NBUF) * NBUF

    def buf(c):
        return rows.at[pl.ds(slot(c) * CHUNK, CHUNK)]

    def gather_start(c):
        return pltpu.async_copy(w_hbm.at[idx_v.at[pl.ds(c * CHUNK, CHUNK)]],
                                buf(c), gsems.at[slot(c)])

    def gather_wait(c):
        pltpu.make_async_copy(w_hbm.at[idx_v.at[pl.ds(c * CHUNK, CHUNK)]],
                              buf(c), gsems.at[slot(c)]).wait()

    def write_start(c):
        def one(b, carry):
            pltpu.async_copy(
                buf(c), out_hbm.at[pl.ds(b * SEQ + seq0 + c * CHUNK, CHUNK)],
                wsems.at[slot(c)])
            return carry
        lax.fori_loop(0, BATCH, one, 0)

    def write_wait(c):
        def one(b, carry):
            pltpu.make_async_copy(
                buf(c), out_hbm.at[pl.ds(b * SEQ + seq0 + c * CHUNK, CHUNK)],
                wsems.at[slot(c)]).wait()
            return carry
        lax.fori_loop(0, BATCH, one, 0)

    def prime(c, carry):
        gather_start(c)
        return carry

    lax.fori_loop(0, NBUF, prime, 0)

    def do_chunk(c, carry):
        gather_wait(c)
        write_start(c)

        @pl.when(c < NCHUNK - NBUF)
        def _refill():
            write_wait(c)
            gather_start(c + NBUF)

        return carry

    lax.fori_loop(0, NCHUNK, do_chunk, 0)

    def drain(c, carry):
        write_wait(NCHUNK - NBUF + c)
        return carry

    lax.fori_loop(0, NBUF, drain, 0)


@functools.partial(
    pl.kernel,
    mesh=_mesh,
    out_type=(),
    scratch_types=[
        pltpu.VMEM((KFIX,), jnp.int32),
        pltpu.VMEM((KFIX,), jnp.int32),
        pltpu.VMEM((KFIX, EMBED_DIM), jnp.float32),
        pltpu.SemaphoreType.DMA,
    ],
)
def _emb_fixup(w_hbm, fixdst_hbm, fixsrc_hbm, out_hbm, fdst_v, fsrc_v,
               fixbuf, fsem):
    wid = lax.axis_index("s") * 2 + lax.axis_index("c")
    pltpu.async_copy(fixdst_hbm.at[pl.ds(wid * KFIX, KFIX)], fdst_v, fsem)
    pltpu.async_copy(fixsrc_hbm.at[pl.ds(wid * KFIX, KFIX)], fsrc_v, fsem)
    pltpu.make_async_copy(fixdst_hbm.at[pl.ds(wid * KFIX, KFIX)],
                          fdst_v, fsem).wait()
    pltpu.make_async_copy(fixsrc_hbm.at[pl.ds(wid * KFIX, KFIX)],
                          fsrc_v, fsem).wait()
    pltpu.async_copy(w_hbm.at[fsrc_v], fixbuf, fsem).wait()
    pltpu.async_copy(fixbuf, out_hbm.at[fdst_v], fsem).wait()


RPW = NROWS // NWORKERS
FCHUNK = 32
FNCHUNK = RPW // FCHUNK
FNPAIR = FNCHUNK // 2


@functools.partial(
    pl.kernel,
    mesh=_mesh,
    out_type=jax.ShapeDtypeStruct((NROWS, EMBED_DIM), jnp.float32),
    scratch_types=[
        pltpu.VMEM((RPW,), jnp.int32),
        pltpu.VMEM((RPW,), jnp.int32),
        pltpu.VMEM((FCHUNK, EMBED_DIM), jnp.float32),
        pltpu.VMEM((FCHUNK, EMBED_DIM), jnp.float32),
        pltpu.SemaphoreType.DMA,
        pltpu.SemaphoreType.DMA,
        pltpu.SemaphoreType.DMA,
        pltpu.SemaphoreType.DMA,
    ],
)
def _emb_gather_full(inp_hbm, w_hbm, out_hbm, inp_v, idx_v, rows0, rows1,
                     gsem0, gsem1, wsem0, wsem1):
    wid = lax.axis_index("s") * 2 + lax.axis_index("c")
    base = wid * RPW

    pltpu.sync_copy(inp_hbm.at[pl.ds(base, RPW)], inp_v)

    def compute_idx(j, carry):
        off = j * LANES
        v = inp_v[pl.ds(off, LANES)]
        s = (base + off + lax.iota(jnp.int32, LANES)) & (SEQ - 1)
        pad_vec = jnp.full((LANES,), PAD, jnp.int32)
        idx_v[pl.ds(off, LANES)] = jnp.where(v != PAD, s + (PAD + 1), pad_vec)
        return carry

    lax.fori_loop(0, RPW // LANES, compute_idx, 0)

    def gather_start(c, rows, gsem):
        return pltpu.async_copy(w_hbm.at[idx_v.at[pl.ds(c * FCHUNK, FCHUNK)]],
                                rows, gsem)

    def gather_wait(c, rows, gsem):
        pltpu.make_async_copy(w_hbm.at[idx_v.at[pl.ds(c * FCHUNK, FCHUNK)]],
                              rows, gsem).wait()

    def write_start(c, rows, wsem):
        pltpu.async_copy(rows, out_hbm.at[pl.ds(base + c * FCHUNK, FCHUNK)],
                         wsem)

    def write_wait(c, rows, wsem):
        pltpu.make_async_copy(rows, out_hbm.at[pl.ds(base + c * FCHUNK, FCHUNK)],
                              wsem).wait()

    gather_start(0, rows0, gsem0)
    gather_start(1, rows1, gsem1)

    def do_pair(g, carry):
        c0 = 2 * g
        c1 = c0 + 1
        gather_wait(c0, rows0, gsem0)
        write_start(c0, rows0, wsem0)
        gather_wait(c1, rows1, gsem1)
        write_start(c1, rows1, wsem1)

        @pl.when(g < FNPAIR - 1)
        def _refill():
            write_wait(c0, rows0, wsem0)
            gather_start(c0 + 2, rows0, gsem0)
            write_wait(c1, rows1, wsem1)
            gather_start(c1 + 2, rows1, gsem1)

        return carry

    lax.fori_loop(0, FNPAIR, do_pair, 0)

    write_wait(FNCHUNK - 2, rows0, wsem0)
    write_wait(FNCHUNK - 1, rows1, wsem1)


def kernel(input, weights):
    inp_flat = input.reshape(-1)

    out1 = _emb_stream(weights)

    padmask = (input == PAD).astype(jnp.int32)
    blocks = padmask.reshape(BATCH, NWORKERS, SPW).transpose(1, 0, 2)
    blocks = blocks.reshape(NWORKERS, BATCH * SPW)
    counts = jnp.sum(blocks, axis=1)
    overflow = jnp.any(counts > KFIX)
    vals, topi = lax.top_k(blocks, KFIX)
    b_loc = topi // SPW
    s_loc = topi - b_loc * SPW
    seq = jnp.arange(NWORKERS, dtype=jnp.int32)[:, None] * SPW + s_loc
    fixdst = (b_loc * SEQ + seq).astype(jnp.int32).reshape(-1)
    fixsrc = jnp.where(vals > 0, PAD, seq + (PAD + 1)).astype(jnp.int32).reshape(-1)

    ref = jax.new_ref(out1)
    _emb_fixup(weights, fixdst, fixsrc, ref)
    out1_fixed = ref[...]

    out = lax.cond(
        overflow,
        lambda o, w, i: _emb_gather_full(i, w),
        lambda o, w, i: o,
        out1_fixed, weights, inp_flat)
    return out.reshape(BATCH, SEQ, EMBED_DIM)


# speedup vs baseline: 1.0078x; 1.0078x over previous
import functools

import jax
import jax.numpy as jnp
from jax import lax
from jax.experimental import pallas as pl
from jax.experimental.pallas import tpu as pltpu
from jax.experimental.pallas import tpu_sc as plsc

EMBED_DIM = 1024
PAD = 1
BATCH = 4
SEQ = 8192
NROWS = BATCH * SEQ
NWORKERS = 32
SPW = SEQ // NWORKERS
CHUNK = 32
NCHUNK = SPW // CHUNK
NBUF = 2
LANES = 16
KFIX = 8

_mesh = plsc.VectorSubcoreMesh(core_axis_name="c", subcore_axis_name="s")


@functools.partial(
    pl.kernel,
    mesh=_mesh,
    out_type=jax.ShapeDtypeStruct((NROWS, EMBED_DIM), jnp.float32),
    scratch_types=[
        pltpu.VMEM((SPW,), jnp.int32),
        pltpu.VMEM((NBUF * CHUNK, EMBED_DIM), jnp.float32),
        pltpu.SemaphoreType.DMA((NBUF,)),
        pltpu.SemaphoreType.DMA((NBUF,)),
    ],
)
def _emb_stream(w_hbm, out_hbm, idx_v, rows, gsems, wsems):
    wid = lax.axis_index("s") * 2 + lax.axis_index("c")
    seq0 = wid * SPW

    def compute_idx(j, carry):
        off = j * LANES
        idx_v[pl.ds(off, LANES)] = (
            seq0 + off + (PAD + 1) + lax.iota(jnp.int32, LANES))
        return carry

    lax.fori_loop(0, SPW // LANES, compute_idx, 0)

    def slot(c):
        return c & (NBUF - 1)

    def buf(c):
        return rows.at[pl.ds(slot(c) * CHUNK, CHUNK)]

    def gather_start(c):
        return pltpu.async_copy(w_hbm.at[idx_v.at[pl.ds(c * CHUNK, CHUNK)]],
                                buf(c), gsems.at[slot(c)])

    def gather_wait(c):
        pltpu.make_async_copy(w_hbm.at[idx_v.at[pl.ds(c * CHUNK, CHUNK)]],
                              buf(c), gsems.at[slot(c)]).wait()

    def write_start(c):
        def one(b, carry):
            pltpu.async_copy(
                buf(c), out_hbm.at[pl.ds(b * SEQ + seq0 + c * CHUNK, CHUNK)],
                wsems.at[slot(c)])
            return carry
        lax.fori_loop(0, BATCH, one, 0)

    def write_wait(c):
        def one(b, carry):
            pltpu.make_async_copy(
                buf(c), out_hbm.at[pl.ds(b * SEQ + seq0 + c * CHUNK, CHUNK)],
                wsems.at[slot(c)]).wait()
            return carry
        lax.fori_loop(0, BATCH, one, 0)

    def prime(c, carry):
        gather_start(c)
        return carry

    lax.fori_loop(0, NBUF, prime, 0)

    def do_chunk(c, carry):
        gather_wait(c)
        write_start(c)

        @pl.when(c < NCHUNK - NBUF)
        def _refill():
            write_wait(c)
            gather_start(c + NBUF)

        return carry

    lax.fori_loop(0, NCHUNK, do_chunk, 0)

    def drain(c, carry):
        write_wait(NCHUNK - NBUF + c)
        return carry

    lax.fori_loop(0, NBUF, drain, 0)


@functools.partial(
    pl.kernel,
    mesh=_mesh,
    out_type=(),
    scratch_types=[
        pltpu.VMEM((KFIX,), jnp.int32),
        pltpu.VMEM((KFIX,), jnp.int32),
        pltpu.VMEM((KFIX, EMBED_DIM), jnp.float32),
        pltpu.SemaphoreType.DMA,
    ],
)
def _emb_fixup(w_hbm, fixdst_hbm, fixsrc_hbm, out_hbm, fdst_v, fsrc_v,
               fixbuf, fsem):
    wid = lax.axis_index("s") * 2 + lax.axis_index("c")
    pltpu.async_copy(fixdst_hbm.at[pl.ds(wid * KFIX, KFIX)], fdst_v, fsem)
    pltpu.async_copy(fixsrc_hbm.at[pl.ds(wid * KFIX, KFIX)], fsrc_v, fsem)
    pltpu.make_async_copy(fixdst_hbm.at[pl.ds(wid * KFIX, KFIX)],
                          fdst_v, fsem).wait()
    pltpu.make_async_copy(fixsrc_hbm.at[pl.ds(wid * KFIX, KFIX)],
                          fsrc_v, fsem).wait()
    pltpu.async_copy(w_hbm.at[fsrc_v], fixbuf, fsem).wait()
    pltpu.async_copy(fixbuf, out_hbm.at[fdst_v], fsem).wait()


RPW = NROWS // NWORKERS
FCHUNK = 32
FNCHUNK = RPW // FCHUNK
FNPAIR = FNCHUNK // 2


@functools.partial(
    pl.kernel,
    mesh=_mesh,
    out_type=jax.ShapeDtypeStruct((NROWS, EMBED_DIM), jnp.float32),
    scratch_types=[
        pltpu.VMEM((RPW,), jnp.int32),
        pltpu.VMEM((RPW,), jnp.int32),
        pltpu.VMEM((FCHUNK, EMBED_DIM), jnp.float32),
        pltpu.VMEM((FCHUNK, EMBED_DIM), jnp.float32),
        pltpu.SemaphoreType.DMA,
        pltpu.SemaphoreType.DMA,
        pltpu.SemaphoreType.DMA,
        pltpu.SemaphoreType.DMA,
    ],
)
def _emb_gather_full(inp_hbm, w_hbm, out_hbm, inp_v, idx_v, rows0, rows1,
                     gsem0, gsem1, wsem0, wsem1):
    wid = lax.axis_index("s") * 2 + lax.axis_index("c")
    base = wid * RPW

    pltpu.sync_copy(inp_hbm.at[pl.ds(base, RPW)], inp_v)

    def compute_idx(j, carry):
        off = j * LANES
        v = inp_v[pl.ds(off, LANES)]
        s = (base + off + lax.iota(jnp.int32, LANES)) & (SEQ - 1)
        pad_vec = jnp.full((LANES,), PAD, jnp.int32)
        idx_v[pl.ds(off, LANES)] = jnp.where(v != PAD, s + (PAD + 1), pad_vec)
        return carry

    lax.fori_loop(0, RPW // LANES, compute_idx, 0)

    def gather_start(c, rows, gsem):
        return pltpu.async_copy(w_hbm.at[idx_v.at[pl.ds(c * FCHUNK, FCHUNK)]],
                                rows, gsem)

    def gather_wait(c, rows, gsem):
        pltpu.make_async_copy(w_hbm.at[idx_v.at[pl.ds(c * FCHUNK, FCHUNK)]],
                              rows, gsem).wait()

    def write_start(c, rows, wsem):
        pltpu.async_copy(rows, out_hbm.at[pl.ds(base + c * FCHUNK, FCHUNK)],
                         wsem)

    def write_wait(c, rows, wsem):
        pltpu.make_async_copy(rows, out_hbm.at[pl.ds(base + c * FCHUNK, FCHUNK)],
                              wsem).wait()

    gather_start(0, rows0, gsem0)
    gather_start(1, rows1, gsem1)

    def do_pair(g, carry):
        c0 = 2 * g
        c1 = c0 + 1
        gather_wait(c0, rows0, gsem0)
        write_start(c0, rows0, wsem0)
        gather_wait(c1, rows1, gsem1)
        write_start(c1, rows1, wsem1)

        @pl.when(g < FNPAIR - 1)
        def _refill():
            write_wait(c0, rows0, wsem0)
            gather_start(c0 + 2, rows0, gsem0)
            write_wait(c1, rows1, wsem1)
            gather_start(c1 + 2, rows1, gsem1)

        return carry

    lax.fori_loop(0, FNPAIR, do_pair, 0)

    write_wait(FNCHUNK - 2, rows0, wsem0)
    write_wait(FNCHUNK - 1, rows1, wsem1)


def kernel(input, weights):
    inp_flat = input.reshape(-1)

    out1 = _emb_stream(weights)

    padmask = (input == PAD).astype(jnp.int32)
    blocks = padmask.reshape(BATCH, NWORKERS, SPW).transpose(1, 0, 2)
    blocks = blocks.reshape(NWORKERS, BATCH * SPW)
    counts = jnp.sum(blocks, axis=1)
    overflow = jnp.any(counts > KFIX)
    vals, topi = lax.top_k(blocks, KFIX)
    b_loc = topi // SPW
    s_loc = topi - b_loc * SPW
    seq = jnp.arange(NWORKERS, dtype=jnp.int32)[:, None] * SPW + s_loc
    fixdst = (b_loc * SEQ + seq).astype(jnp.int32).reshape(-1)
    fixsrc = jnp.where(vals > 0, PAD, seq + (PAD + 1)).astype(jnp.int32).reshape(-1)

    ref = jax.new_ref(out1)
    _emb_fixup(weights, fixdst, fixsrc, ref)
    out1_fixed = ref[...]

    out = lax.cond(
        overflow,
        lambda o, w, i: _emb_gather_full(i, w),
        lambda o, w, i: o,
        out1_fixed, weights, inp_flat)
    return out.reshape(BATCH, SEQ, EMBED_DIM)
